# Initial kernel scaffold; baseline (speedup 1.0000x reference)
import jax
import jax.numpy as jnp
from jax.experimental import pallas as pl


def kernel(x_scalar, season_q, year_q, memory_bank, memory_seasons, memory_years, params):
    raise NotImplementedError("write your pallas kernel here")


# speedup vs baseline: 2.2831x; 2.2831x over previous
import functools

import jax
import jax.numpy as jnp
import numpy as np
from jax.experimental import pallas as pl
from jax.experimental.pallas import tpu as pltpu

B, T, N, D = 64, 96, 128, 128
M, K = 1196, 8
H, DH = 4, 32
OB = 512
MB = 128
NMB = 10


def _gelu_exact(x):
    sqrt_half = np.float32(np.sqrt(0.5))
    return 0.5 * x * (1.0 + jax.lax.erf(x * sqrt_half))


def _conv_kernel(xt_ref, dww_ref, dwb_ref, h_ref):
    xt = xt_ref[...]
    acc = xt[0:97] * dww_ref[0, :][None, None, :]
    for j in range(1, 12):
        acc = acc + xt[j:j + 97] * dww_ref[j, :][None, None, :]
    acc = acc + dwb_ref[0, :][None, None, :]
    h_ref[...] = _gelu_exact(acc)


def _pw_kernel(pw_ref, pwb_ref, h_ref, g_ref, b_ref, q_ref):
    mm = jnp.dot(pw_ref[...], h_ref[...])
    mm = mm + pwb_ref[...]
    g = _gelu_exact(mm)
    acc128 = g[:, 0:128]
    for j in range(1, 48):
        acc128 = acc128 + g[:, 128 * j:128 * (j + 1)]
    acc = acc128[:, 0:64] + acc128[:, 64:128] + g[:, 6144:6208]
    qm = acc / 97.0
    q3 = qm.reshape(OB // 128, 128, 64)
    mu = jnp.mean(q3, axis=1, keepdims=True)
    var = jnp.mean((q3 - mu) ** 2, axis=1, keepdims=True)
    qn = (q3 - mu) / jnp.sqrt(var + 1e-5) * g_ref[...][None] + b_ref[...][None]
    q_ref[...] = qn.reshape(OB, 64)


def _sim_topk_kernel(sq_ref, yq_ref, mem_ref, sm_ref, ym_ref, q_ref, idx_ref,
                     qn_scr, sim_scr):
    i = pl.program_id(0)

    @pl.when(i == 0)
    def _():
        q = q_ref[...]
        nq = jnp.sqrt(jnp.sum(q * q, axis=0, keepdims=True))
        qn_scr[...] = q / jnp.maximum(nq, 1e-12)

    mem = mem_ref[...]
    nm = jnp.sqrt(jnp.sum(mem * mem, axis=1, keepdims=True))
    memn = mem / jnp.maximum(nm, 1e-12)
    simb = jnp.dot(memn, qn_scr[...])
    smask = (sq_ref[...] == sm_ref[...]).astype(jnp.float32)
    simb = simb * smask + (1.0 - smask) * (-10000.0)
    dy = jnp.abs(yq_ref[...] - ym_ref[...])
    simb = simb * (0.5 + 0.5 * (1.0 - jnp.exp(-dy / 2.0)))
    sim_scr[i] = simb

    @pl.when(i == NMB - 1)
    def _():
        s = sim_scr[...]
        i0 = jax.lax.broadcasted_iota(jnp.int32, (NMB, MB, 64), 0)
        i1 = jax.lax.broadcasted_iota(jnp.int32, (NMB, MB, 64), 1)
        midx = i0 * MB + i1
        neg = jnp.float32(-3e38)
        cur = jnp.where(midx < M, s, neg)
        for k in range(K):
            mx = jnp.max(cur, axis=(0, 1), keepdims=True)
            cand = jnp.where(cur >= mx, midx, jnp.int32(2 ** 30))
            sel = jnp.min(cand, axis=(0, 1), keepdims=True)
            idx_ref[k, :] = sel[0, 0]
            cur = jnp.where(midx == sel, neg, cur)


def _attn_kernel(idx_ref, q_ref, mem_hbm, wq_ref, bq_ref, wkv_ref, bkv_ref,
                 wo_ref, bo_ref, wp_ref, bp_ref, out_ref, kv_scr, sems):
    b = pl.program_id(0)
    for k in range(K):
        pltpu.make_async_copy(mem_hbm.at[idx_ref[b * K + k]], kv_scr.at[k],
                              sems.at[k]).start()
    for k in range(K):
        pltpu.make_async_copy(mem_hbm.at[idx_ref[b * K + k]], kv_scr.at[k],
                              sems.at[k]).wait()
    kv = kv_scr[...].reshape(K * N, D)
    kvp = jnp.dot(kv, wkv_ref[...]) + bkv_ref[...]
    kp = kvp[:, 0:D].reshape(K, N, D)
    vp = kvp[:, D:2 * D].reshape(K, N, D)
    qp = jnp.dot(q_ref[0], wq_ref[...]) + bq_ref[...]
    prod = qp[None, :, :] * kp
    outs = []
    for h in range(H):
        sl = slice(DH * h, DH * (h + 1))
        s_h = jnp.sum(prod[:, :, sl], axis=-1) / np.float32(np.sqrt(DH))
        mx = jnp.max(s_h, axis=0, keepdims=True)
        e = jnp.exp(s_h - mx)
        att = e / jnp.sum(e, axis=0, keepdims=True)
        outs.append(jnp.sum(att[:, :, None] * vp[:, :, sl], axis=0))
    o = jnp.concatenate(outs, axis=-1)
    o1 = jnp.dot(o, wo_ref[...]) + bo_ref[...]
    out_ref[0] = jnp.dot(o1, wp_ref[...]) + bp_ref[...]


@jax.jit
def kernel(x_scalar, season_q, year_q, memory_bank, memory_seasons,
           memory_years, params):
    p = params
    xt = jnp.pad(jnp.transpose(x_scalar, (1, 0, 2)), ((6, 6), (0, 0), (0, 0)))
    dww = jnp.transpose(p['dw_w'].reshape(N, 12))
    dwb = p['dw_b'][None, :]
    h3 = pl.pallas_call(
        _conv_kernel,
        out_shape=jax.ShapeDtypeStruct((97, B, N), jnp.float32),
    )(xt, dww, dwb)
    h_r = jnp.transpose(h3, (2, 0, 1)).reshape(N, 97 * B)

    q2 = pl.pallas_call(
        _pw_kernel,
        grid=(N * D // OB,),
        in_specs=[
            pl.BlockSpec((OB, N), lambda i: (i, 0)),
            pl.BlockSpec((OB, 1), lambda i: (i, 0)),
            pl.BlockSpec((N, 97 * B), lambda i: (0, 0)),
            pl.BlockSpec((D, 1), lambda i: (0, 0)),
            pl.BlockSpec((D, 1), lambda i: (0, 0)),
        ],
        out_specs=pl.BlockSpec((OB, 64), lambda i: (i, 0)),
        out_shape=jax.ShapeDtypeStruct((N * D, B), jnp.float32),
    )(p['pw_w'], p['pw_b'][:, None], h_r, p['ln_g'][:, None], p['ln_b'][:, None])
    q_out = jnp.transpose(q2).reshape(B, N, D)

    mem2 = memory_bank.reshape(M, N * D)
    topk_t = pl.pallas_call(
        _sim_topk_kernel,
        grid=(NMB,),
        in_specs=[
            pl.BlockSpec((1, B), lambda i: (0, 0)),
            pl.BlockSpec((1, B), lambda i: (0, 0)),
            pl.BlockSpec((MB, N * D), lambda i: (i, 0)),
            pl.BlockSpec((MB, 1), lambda i: (i, 0)),
            pl.BlockSpec((MB, 1), lambda i: (i, 0)),
            pl.BlockSpec((N * D, B), lambda i: (0, 0)),
        ],
        out_specs=pl.BlockSpec((K, B), lambda i: (0, 0)),
        out_shape=jax.ShapeDtypeStruct((K, B), jnp.int32),
        scratch_shapes=[
            pltpu.VMEM((N * D, B), jnp.float32),
            pltpu.VMEM((NMB, MB, B), jnp.float32),
        ],
    )(season_q[None, :].astype(jnp.int32), year_q[None, :], mem2,
      memory_seasons[:, None].astype(jnp.int32), memory_years[:, None], q2)
    idx_flat = jnp.transpose(topk_t).reshape(-1)

    w_in = p['w_in']
    z = pl.pallas_call(
        _attn_kernel,
        grid_spec=pltpu.PrefetchScalarGridSpec(
            num_scalar_prefetch=1,
            grid=(B,),
            in_specs=[
                pl.BlockSpec((1, N, D), lambda b, idx: (b, 0, 0)),
                pl.BlockSpec(memory_space=pl.ANY),
                pl.BlockSpec((D, D), lambda b, idx: (0, 0)),
                pl.BlockSpec((1, D), lambda b, idx: (0, 0)),
                pl.BlockSpec((D, 2 * D), lambda b, idx: (0, 0)),
                pl.BlockSpec((1, 2 * D), lambda b, idx: (0, 0)),
                pl.BlockSpec((D, D), lambda b, idx: (0, 0)),
                pl.BlockSpec((1, D), lambda b, idx: (0, 0)),
                pl.BlockSpec((D, D), lambda b, idx: (0, 0)),
                pl.BlockSpec((1, D), lambda b, idx: (0, 0)),
            ],
            out_specs=pl.BlockSpec((1, N, D), lambda b, idx: (b, 0, 0)),
            scratch_shapes=[
                pltpu.VMEM((K, N, D), jnp.float32),
                pltpu.SemaphoreType.DMA((K,)),
            ],
        ),
        out_shape=jax.ShapeDtypeStruct((B, N, D), jnp.float32),
    )(idx_flat, q_out, memory_bank,
      jnp.transpose(w_in[0:D]), p['b_in'][None, 0:D],
      jnp.transpose(w_in[D:3 * D]), p['b_in'][None, D:3 * D],
      jnp.transpose(p['w_out']), p['b_out'][None],
      jnp.transpose(p['w_proj']), p['b_proj'][None])
    return z, q_out


# speedup vs baseline: 2.3206x; 1.0164x over previous
import functools

import jax
import jax.numpy as jnp
import numpy as np
from jax.experimental import pallas as pl
from jax.experimental.pallas import tpu as pltpu

B, T, N, D = 64, 96, 128, 128
M, K = 1196, 8
H, DH = 4, 32
OB = 512
MB = 128
NMB = 10


def _gelu_exact(x):
    sqrt_half = np.float32(np.sqrt(0.5))
    return 0.5 * x * (1.0 + jax.lax.erf(x * sqrt_half))


def _conv_kernel(xt_ref, dww_ref, dwb_ref, h_ref):
    xt = xt_ref[...]
    acc = xt[0:97] * dww_ref[0, :][None, None, :]
    for j in range(1, 12):
        acc = acc + xt[j:j + 97] * dww_ref[j, :][None, None, :]
    acc = acc + dwb_ref[0, :][None, None, :]
    g = _gelu_exact(acc)
    h_ref[...] = jnp.transpose(g.reshape(97 * B, N), (1, 0))


def _pw_kernel(pw_ref, pwb_ref, h_ref, g_ref, b_ref, q_ref, qt_ref):
    mm = jnp.dot(pw_ref[...], h_ref[...])
    mm = mm + pwb_ref[...]
    g = _gelu_exact(mm)
    acc128 = g[:, 0:128]
    for j in range(1, 48):
        acc128 = acc128 + g[:, 128 * j:128 * (j + 1)]
    acc = acc128[:, 0:64] + acc128[:, 64:128] + g[:, 6144:6208]
    qm = acc / 97.0
    q3 = qm.reshape(OB // 128, 128, 64)
    mu = jnp.mean(q3, axis=1, keepdims=True)
    var = jnp.mean((q3 - mu) ** 2, axis=1, keepdims=True)
    qn = (q3 - mu) / jnp.sqrt(var + 1e-5) * g_ref[...][None] + b_ref[...][None]
    qn = qn.reshape(OB, 64)
    q_ref[...] = qn
    qt_ref[...] = jnp.transpose(qn, (1, 0))


def _sim_topk_kernel(sq_ref, yq_ref, mem_ref, sm_ref, ym_ref, q_ref, idx_ref,
                     qn_scr, sim_scr):
    i = pl.program_id(0)

    @pl.when(i == 0)
    def _():
        q = q_ref[...]
        nq = jnp.sqrt(jnp.sum(q * q, axis=0, keepdims=True))
        qn_scr[...] = q / jnp.maximum(nq, 1e-12)

    mem = mem_ref[...]
    nm = jnp.sqrt(jnp.sum(mem * mem, axis=1, keepdims=True))
    memn = mem / jnp.maximum(nm, 1e-12)
    simb = jnp.dot(memn, qn_scr[...])
    smask = (sq_ref[...] == sm_ref[...]).astype(jnp.float32)
    simb = simb * smask + (1.0 - smask) * (-10000.0)
    dy = jnp.abs(yq_ref[...] - ym_ref[...])
    simb = simb * (0.5 + 0.5 * (1.0 - jnp.exp(-dy / 2.0)))
    sim_scr[i] = simb

    @pl.when(i == NMB - 1)
    def _():
        s = sim_scr[...]
        i0 = jax.lax.broadcasted_iota(jnp.int32, (NMB, MB, 64), 0)
        i1 = jax.lax.broadcasted_iota(jnp.int32, (NMB, MB, 64), 1)
        midx = i0 * MB + i1
        neg = jnp.float32(-3e38)
        cur = jnp.where(midx < M, s, neg)
        for k in range(K):
            mx = jnp.max(cur, axis=(0, 1), keepdims=True)
            cand = jnp.where(cur >= mx, midx, jnp.int32(2 ** 30))
            sel = jnp.min(cand, axis=(0, 1), keepdims=True)
            idx_ref[k, :] = sel[0, 0]
            cur = jnp.where(midx == sel, neg, cur)


def _attn_kernel(idx_ref, q_ref, mem_hbm, wq_ref, bq_ref, wkv_ref, bkv_ref,
                 wo_ref, bo_ref, wp_ref, bp_ref, out_ref, kv_scr, sems):
    b = pl.program_id(0)
    for k in range(K):
        pltpu.make_async_copy(mem_hbm.at[idx_ref[b * K + k]], kv_scr.at[k],
                              sems.at[k]).start()
    for k in range(K):
        pltpu.make_async_copy(mem_hbm.at[idx_ref[b * K + k]], kv_scr.at[k],
                              sems.at[k]).wait()
    kv = kv_scr[...].reshape(K * N, D)
    kvp = jnp.dot(kv, wkv_ref[...]) + bkv_ref[...]
    kp = kvp[:, 0:D].reshape(K, N, D)
    vp = kvp[:, D:2 * D].reshape(K, N, D)
    qp = jnp.dot(q_ref[0], wq_ref[...]) + bq_ref[...]
    prod = qp[None, :, :] * kp
    outs = []
    for h in range(H):
        sl = slice(DH * h, DH * (h + 1))
        s_h = jnp.sum(prod[:, :, sl], axis=-1) / np.float32(np.sqrt(DH))
        mx = jnp.max(s_h, axis=0, keepdims=True)
        e = jnp.exp(s_h - mx)
        att = e / jnp.sum(e, axis=0, keepdims=True)
        outs.append(jnp.sum(att[:, :, None] * vp[:, :, sl], axis=0))
    o = jnp.concatenate(outs, axis=-1)
    o1 = jnp.dot(o, wo_ref[...]) + bo_ref[...]
    out_ref[0] = jnp.dot(o1, wp_ref[...]) + bp_ref[...]


@jax.jit
def kernel(x_scalar, season_q, year_q, memory_bank, memory_seasons,
           memory_years, params):
    p = params
    xt = jnp.pad(jnp.transpose(x_scalar, (1, 0, 2)), ((6, 6), (0, 0), (0, 0)))
    dww = jnp.transpose(p['dw_w'].reshape(N, 12))
    dwb = p['dw_b'][None, :]
    h_r = pl.pallas_call(
        _conv_kernel,
        out_shape=jax.ShapeDtypeStruct((N, 97 * B), jnp.float32),
    )(xt, dww, dwb)

    q2 = pl.pallas_call(
        _pw_kernel,
        grid=(N * D // OB,),
        in_specs=[
            pl.BlockSpec((OB, N), lambda i: (i, 0)),
            pl.BlockSpec((OB, 1), lambda i: (i, 0)),
            pl.BlockSpec((N, 97 * B), lambda i: (0, 0)),
            pl.BlockSpec((D, 1), lambda i: (0, 0)),
            pl.BlockSpec((D, 1), lambda i: (0, 0)),
        ],
        out_specs=[pl.BlockSpec((OB, 64), lambda i: (i, 0)),
                   pl.BlockSpec((B, OB), lambda i: (0, i))],
        out_shape=[jax.ShapeDtypeStruct((N * D, B), jnp.float32),
                   jax.ShapeDtypeStruct((B, N * D), jnp.float32)],
    )(p['pw_w'], p['pw_b'][:, None], h_r, p['ln_g'][:, None], p['ln_b'][:, None])
    q2, qt = q2
    q_out = qt.reshape(B, N, D)

    mem2 = memory_bank.reshape(M, N * D)
    topk_t = pl.pallas_call(
        _sim_topk_kernel,
        grid=(NMB,),
        in_specs=[
            pl.BlockSpec((1, B), lambda i: (0, 0)),
            pl.BlockSpec((1, B), lambda i: (0, 0)),
            pl.BlockSpec((MB, N * D), lambda i: (i, 0)),
            pl.BlockSpec((MB, 1), lambda i: (i, 0)),
            pl.BlockSpec((MB, 1), lambda i: (i, 0)),
            pl.BlockSpec((N * D, B), lambda i: (0, 0)),
        ],
        out_specs=pl.BlockSpec((K, B), lambda i: (0, 0)),
        out_shape=jax.ShapeDtypeStruct((K, B), jnp.int32),
        scratch_shapes=[
            pltpu.VMEM((N * D, B), jnp.float32),
            pltpu.VMEM((NMB, MB, B), jnp.float32),
        ],
    )(season_q[None, :].astype(jnp.int32), year_q[None, :], mem2,
      memory_seasons[:, None].astype(jnp.int32), memory_years[:, None], q2)
    idx_flat = jnp.transpose(topk_t).reshape(-1)

    w_in = p['w_in']
    z = pl.pallas_call(
        _attn_kernel,
        grid_spec=pltpu.PrefetchScalarGridSpec(
            num_scalar_prefetch=1,
            grid=(B,),
            in_specs=[
                pl.BlockSpec((1, N, D), lambda b, idx: (b, 0, 0)),
                pl.BlockSpec(memory_space=pl.ANY),
                pl.BlockSpec((D, D), lambda b, idx: (0, 0)),
                pl.BlockSpec((1, D), lambda b, idx: (0, 0)),
                pl.BlockSpec((D, 2 * D), lambda b, idx: (0, 0)),
                pl.BlockSpec((1, 2 * D), lambda b, idx: (0, 0)),
                pl.BlockSpec((D, D), lambda b, idx: (0, 0)),
                pl.BlockSpec((1, D), lambda b, idx: (0, 0)),
                pl.BlockSpec((D, D), lambda b, idx: (0, 0)),
                pl.BlockSpec((1, D), lambda b, idx: (0, 0)),
            ],
            out_specs=pl.BlockSpec((1, N, D), lambda b, idx: (b, 0, 0)),
            scratch_shapes=[
                pltpu.VMEM((K, N, D), jnp.float32),
                pltpu.SemaphoreType.DMA((K,)),
            ],
        ),
        out_shape=jax.ShapeDtypeStruct((B, N, D), jnp.float32),
    )(idx_flat, q_out, memory_bank,
      jnp.transpose(w_in[0:D]), p['b_in'][None, 0:D],
      jnp.transpose(w_in[D:3 * D]), p['b_in'][None, D:3 * D],
      jnp.transpose(p['w_out']), p['b_out'][None],
      jnp.transpose(p['w_proj']), p['b_proj'][None])
    return z, q_out
